# baseline (speedup 1.0000x reference)
import dataclasses

import jax
import jax.numpy as jnp
from jax import lax
from jax.experimental import pallas as pl
from jax.experimental.pallas import tpu as pltpu
from jax.experimental.pallas import tpu_sc as plsc

_TOTAL_AMOUNT = 30000000.0
_MIN_AMOUNT = 1500.0
_MIN_RATIO = _MIN_AMOUNT / _TOTAL_AMOUNT
_P = 16
_N = 65536
_HALF = _N // 2
_RANK_A = _N // 2 - 1
_BIG = 0x7FFFFFFF
_SENT = 0x40000000
_L = 16
_NB = 256
_UNROLL = 8
_CAPL = 256
_CAP = _L * _CAPL
_GLO = 124
_GHI = 131
_ROUNDS = ((22, None), (14, 22), (6, 14), (0, 6))


def _sc_body(*refs):
    xs = refs[:_P]
    (o_hbm, data_v, cand_v, pcand_v, hist_v, comb_v, tmp_v, shared_v,
     shcand_v, minx_v, out_v) = refs[_P:]
    c = lax.axis_index("c")
    s = lax.axis_index("s")
    proj = c * 8 + (s // 2)
    half = s & 1

    iota = lax.iota(jnp.int32, _L)
    lane_off = iota * _NB
    ones = jnp.ones((_L,), jnp.int32)
    sent_vec = jnp.full((_L,), _SENT, jnp.int32)

    for k in range(_P):
        @pl.when(proj == k)
        def _(k=k):
            pltpu.sync_copy(xs[k].at[pl.ds(half * _HALF, _HALF)], data_v)

    def bits_at(off):
        return plsc.bitcast(data_v[pl.ds(off, _L)], jnp.int32)

    def vbucket_at(off):
        return (data_v[pl.ds(off, _L)] * float(_NB)).astype(jnp.int32)

    def zero_hist():
        @pl.loop(0, _NB * _L, step=_L)
        def _(i):
            hist_v[pl.ds(i, _L)] = jnp.zeros((_L,), jnp.int32)

    def combine():
        @pl.loop(0, _L)
        def _(si):
            acc = hist_v[pl.ds(si * _L, _L)]
            for ci in range(1, _L):
                acc = acc + hist_v[pl.ds(ci * _NB + si * _L, _L)]
            comb_v[pl.ds(si * _L, _L)] = acc

    def exchange():
        pltpu.sync_copy(comb_v, shared_v.at[s])
        plsc.subcore_barrier()
        pltpu.sync_copy(shared_v.at[s ^ 1], tmp_v)
        @pl.loop(0, _L)
        def _(si):
            comb_v[pl.ds(si * _L, _L)] = (comb_v[pl.ds(si * _L, _L)]
                                          + tmp_v[pl.ds(si * _L, _L)])
        plsc.subcore_barrier()

    def select(target):
        g_tot = plsc.load_gather(comb_v, [iota * _L])
        for k in range(1, _L):
            g_tot = g_tot + plsc.load_gather(comb_v, [iota * _L + k])
        gp = jnp.cumsum(g_tot)
        gstar = jnp.min(jnp.where(gp >= target, iota, _L))
        base = jnp.sum(jnp.where(iota < gstar, g_tot, 0))
        h = plsc.load_gather(comb_v, [gstar * _L + iota])
        wp = jnp.cumsum(h) + base
        jstar = jnp.min(jnp.where(wp >= target, iota, _L))
        nb = base + jnp.sum(jnp.where(iota < jstar, h, 0))
        hj = jnp.sum(jnp.where(iota == jstar, h, 0))
        return gstar * _L + jstar, nb, hj

    def radix_rounds(scan_round, target0):
        prefix = jnp.int32(0)
        target = target0
        hj = jnp.int32(0)
        for sh, msh in _ROUNDS:
            zero_hist()
            scan_round(sh, msh, prefix)
            b, nb, hj = select(target)
            prefix = b if msh is None else ((prefix << (msh - sh)) | b)
            target = target - nb
        return prefix, (_RANK_A + 1 - target) + hj

    def emit_result(va, vb):
        out_v[...] = plsc.bitcast(
            jnp.where(iota == 0, va, jnp.where(iota == 1, vb, 0)),
            jnp.float32)

    lane_base = iota * _CAPL
    clamp_vec = lane_base + (_CAPL - 1)
    zero_vec = jnp.zeros((_L,), jnp.int32)
    carry0 = (lane_base, zero_vec, zero_vec)

    @plsc.parallel_loop(0, _HALF, _L, unroll=_UNROLL, carry=carry0)
    def fused(c0, carry):
        ptrs, cntacc, lo_acc = carry
        vb = vbucket_at(c0)
        m_lo = vb < _GLO
        m_mid = (vb >= _GLO) & (vb <= _GHI)
        plsc.store_scatter(cand_v, [ptrs], bits_at(c0), mask=m_mid)
        lo_acc = lo_acc + m_lo.astype(jnp.int32)
        cntacc = cntacc + m_mid.astype(jnp.int32)
        ptrs = jnp.minimum(ptrs + m_mid.astype(jnp.int32), clamp_vec)
        return ptrs, cntacc, lo_acc

    _ptrs, cntacc, lo_acc = fused
    cnt_lo = jnp.sum(lo_acc)
    nmax_o = jnp.max(cntacc)

    minx_v[...] = cntacc
    pltpu.sync_copy(minx_v, shared_v.at[s, pl.ds(0, _L)])
    minx_v[...] = jnp.where(iota == 0, cnt_lo, 0)
    pltpu.sync_copy(minx_v, shared_v.at[s, pl.ds(_L, _L)])
    pltpu.sync_copy(cand_v.at[pl.ds(0, _CAP)], shcand_v.at[s])
    plsc.subcore_barrier()
    pltpu.sync_copy(shared_v.at[s ^ 1, pl.ds(0, 2 * _L)],
                    tmp_v.at[pl.ds(0, 2 * _L)])
    pcnts = tmp_v[pl.ds(0, _L)]
    aux = tmp_v[pl.ds(_L, _L)]
    cnt_lo_par = jnp.sum(jnp.where(iota == 0, aux, 0))
    nmax_p = jnp.max(pcnts)
    cnt_lo_g = cnt_lo + cnt_lo_par
    good = ((cnt_lo_g <= _RANK_A)
            & ((cnt_lo_g + jnp.sum(cntacc) + jnp.sum(pcnts)) >= _RANK_A + 2)
            & (nmax_o <= _CAPL - 1) & (nmax_p <= _CAPL - 1))

    @pl.when(good)
    def _():
        pltpu.sync_copy(shcand_v.at[s ^ 1], pcand_v.at[pl.ds(0, _CAP)])
        bufs = ((cand_v, cntacc, nmax_o), (pcand_v, pcnts, nmax_p))

        def scan_round(sh, msh, prefix):
            for buf, cnts, nmax in bufs:
                @plsc.parallel_loop(0, nmax, 1, unroll=4)
                def _(j, buf=buf, cnts=cnts):
                    v = plsc.load_gather(buf, [lane_base + j])
                    m = cnts > j
                    if msh is not None:
                        m = m & ((v >> msh) == prefix)
                    bucket = (v >> sh) & (0x3F if sh == 0 else 0xFF)
                    plsc.addupdate_scatter(hist_v, [lane_off + bucket],
                                           ones, mask=m)
            combine()

        va, cnt_le_a = radix_rounds(scan_round, _RANK_A + 1 - cnt_lo_g)

        mn = jnp.full((_L,), _BIG, jnp.int32)
        for buf, cnts, nmax in bufs:
            @plsc.parallel_loop(0, nmax, 1, unroll=4, carry=mn)
            def mloop(j, acc, buf=buf, cnts=cnts):
                v = plsc.load_gather(buf, [lane_base + j])
                keep = (cnts > j) & (v > va)
                return jnp.minimum(acc, jnp.where(keep, v, _BIG))
            mn = mloop
        min_above = jnp.min(mn)
        emit_result(va, jnp.where(cnt_le_a >= _RANK_A + 2, va, min_above))

        for _i in range(11):
            plsc.subcore_barrier()

    @pl.when(jnp.logical_not(good))
    def _():
        zero_hist()

        @pl.loop(0, _HALF, step=_L * _UNROLL)
        def _(c0):
            for j in range(_UNROLL):
                plsc.addupdate_scatter(
                    hist_v, [lane_off + vbucket_at(c0 + j * _L)], ones)

        combine()
        exchange()
        ba, nba, _u = select(_RANK_A + 1)
        bb, _u2, _u3 = select(_RANK_A + 2)

        @plsc.parallel_loop(0, _HALF, _L, unroll=_UNROLL, carry=jnp.int32(0))
        def compact(c0, off):
            b = vbucket_at(c0)
            m = (b == ba) | (b == bb)
            plsc.store_compressed(cand_v.at[pl.ds(off, _L)], bits_at(c0),
                                  mask=m)
            return off + jnp.max(plsc.all_reduce_population_count(m))

        fcnt = compact
        cand_v[pl.ds(fcnt, _L)] = sent_vec
        nsl = (fcnt + _L - 1) >> 4

        def scan_round(sh, msh, prefix):
            @pl.loop(0, nsl)
            def _(i):
                v = cand_v[pl.ds(i * _L, _L)]
                vf = plsc.bitcast(v, jnp.float32)
                m = (vf * float(_NB)).astype(jnp.int32) == ba
                if msh is not None:
                    m = m & ((v >> msh) == prefix)
                bucket = (v >> sh) & (0x3F if sh == 0 else 0xFF)
                plsc.addupdate_scatter(hist_v, [lane_off + bucket], ones,
                                       mask=m)
            combine()
            exchange()

        va, cnt_le_a = radix_rounds(scan_round, _RANK_A + 1 - nba)

        minx_v[...] = jnp.full((_L,), _BIG, jnp.int32)

        @pl.loop(0, nsl)
        def _(i):
            v = cand_v[pl.ds(i * _L, _L)]
            minx_v[...] = jnp.minimum(minx_v[...],
                                      jnp.where(v > va, v, _BIG))

        pltpu.sync_copy(minx_v, shared_v.at[s, pl.ds(0, _L)])
        plsc.subcore_barrier()
        pltpu.sync_copy(shared_v.at[s ^ 1, pl.ds(0, _L)],
                        tmp_v.at[pl.ds(0, _L)])
        min_above = jnp.min(jnp.minimum(minx_v[...], tmp_v[pl.ds(0, _L)]))
        emit_result(va, jnp.where(cnt_le_a >= _RANK_A + 2, va, min_above))

    @pl.when(half == 0)
    def _():
        pltpu.sync_copy(out_v, o_hbm.at[proj])


def _epilogue_body(r_ref, o_ref):
    ceil_v = r_ref[:, 0:1]
    floor_v = r_ref[:, 1:2]
    median = (ceil_v + floor_v) * 0.5
    scaled_min = ceil_v * _MIN_RATIO
    sms = jnp.sum(scaled_min)
    meets_min = (median >= sms).astype(jnp.float32)
    rescaled = _MIN_AMOUNT * (median / sms) * meets_min
    votes = jnp.full((_P, 1), float(_N), jnp.float32)
    elig = jnp.ones((_P, 1), jnp.float32)
    o_ref[...] = jnp.concatenate([votes, median, elig, rescaled], axis=1)


def kernel(x0, x1, x2, x3, x4, x5, x6, x7, x8, x9, x10, x11, x12, x13, x14, x15):
    cp = pltpu.CompilerParams()
    if "needs_layout_passes" in pltpu.CompilerParams.__dataclass_fields__:
        cp = dataclasses.replace(cp, needs_layout_passes=False)
    sc_fn = pl.kernel(
        _sc_body,
        out_type=jax.ShapeDtypeStruct((_P, _L), jnp.float32),
        mesh=plsc.VectorSubcoreMesh(core_axis_name="c", subcore_axis_name="s"),
        compiler_params=cp,
        scratch_types=[
            pltpu.VMEM((_HALF,), jnp.float32),
            pltpu.VMEM((_HALF + 2 * _L,), jnp.int32),
            pltpu.VMEM((_CAP + 2 * _L,), jnp.int32),
            pltpu.VMEM((_NB * _L,), jnp.int32),
            pltpu.VMEM((_NB,), jnp.int32),
            pltpu.VMEM((_NB,), jnp.int32),
            pltpu.VMEM_SHARED((_L, _NB), jnp.int32),
            pltpu.VMEM_SHARED((_L, _CAP), jnp.int32),
            pltpu.VMEM((_L,), jnp.int32),
            pltpu.VMEM((_L,), jnp.float32),
        ],
    )
    r = sc_fn(x0, x1, x2, x3, x4, x5, x6, x7, x8, x9, x10, x11, x12, x13,
              x14, x15)

    return pl.pallas_call(
        _epilogue_body,
        out_shape=jax.ShapeDtypeStruct((_P, 4), jnp.float32),
        in_specs=[pl.BlockSpec(memory_space=pltpu.VMEM)],
        out_specs=pl.BlockSpec(memory_space=pltpu.VMEM),
    )(r)


# speedup vs baseline: 1.1602x; 1.1602x over previous
import dataclasses

import jax
import jax.numpy as jnp
from jax import lax
from jax.experimental import pallas as pl
from jax.experimental.pallas import tpu as pltpu
from jax.experimental.pallas import tpu_sc as plsc

_TOTAL_AMOUNT = 30000000.0
_MIN_AMOUNT = 1500.0
_MIN_RATIO = _MIN_AMOUNT / _TOTAL_AMOUNT
_P = 16
_N = 65536
_HALF = _N // 2
_RANK_A = _N // 2 - 1
_BIG = 0x7FFFFFFF
_L = 16
_NB = 256
_UNROLL = 8


def _sc_body(*refs):
    xs = refs[:_P]
    (o_hbm, data_v, cand_v, hist_v, comb_v, tmp_v, shared_v, minx_v,
     out_v) = refs[_P:]
    c = lax.axis_index("c")
    s = lax.axis_index("s")
    proj = c * 8 + (s // 2)
    half = s & 1

    iota = lax.iota(jnp.int32, _L)
    lane_off = iota * _NB
    ones = jnp.ones((_L,), jnp.int32)

    for k in range(_P):
        @pl.when(proj == k)
        def _(k=k):
            pltpu.sync_copy(xs[k].at[pl.ds(half * _HALF, _HALF)], data_v)

    def bits_at(off):
        return plsc.bitcast(data_v[pl.ds(off, _L)], jnp.int32)

    def zero_hist():
        @pl.loop(0, _NB * _L, step=_L)
        def _(i):
            hist_v[pl.ds(i, _L)] = jnp.zeros((_L,), jnp.int32)

    def combine_and_exchange():
        @pl.loop(0, _L)
        def _(si):
            acc = hist_v[pl.ds(si * _L, _L)]
            for ci in range(1, _L):
                acc = acc + hist_v[pl.ds(ci * _NB + si * _L, _L)]
            comb_v[pl.ds(si * _L, _L)] = acc
        pltpu.sync_copy(comb_v, shared_v.at[s])
        plsc.subcore_barrier()
        pltpu.sync_copy(shared_v.at[s ^ 1], tmp_v)
        @pl.loop(0, _L)
        def _(si):
            comb_v[pl.ds(si * _L, _L)] = (comb_v[pl.ds(si * _L, _L)]
                                          + tmp_v[pl.ds(si * _L, _L)])
        plsc.subcore_barrier()

    def select(target):
        g_tot = plsc.load_gather(comb_v, [iota * _L])
        for k in range(1, _L):
            g_tot = g_tot + plsc.load_gather(comb_v, [iota * _L + k])
        gp = jnp.cumsum(g_tot)
        gstar = jnp.min(jnp.where(gp >= target, iota, _L))
        base = jnp.sum(jnp.where(iota < gstar, g_tot, 0))
        h = plsc.load_gather(comb_v, [gstar * _L + iota])
        wp = jnp.cumsum(h) + base
        jstar = jnp.min(jnp.where(wp >= target, iota, _L))
        nb = base + jnp.sum(jnp.where(iota < jstar, h, 0))
        hj = jnp.sum(jnp.where(iota == jstar, h, 0))
        return gstar * _L + jstar, nb, hj

    zero_hist()

    def vbucket_at(off):
        return (data_v[pl.ds(off, _L)] * float(_NB)).astype(jnp.int32)

    @plsc.parallel_loop(0, _HALF, _L, unroll=_UNROLL)
    def _(c0):
        plsc.addupdate_scatter(hist_v, [lane_off + vbucket_at(c0)], ones)

    combine_and_exchange()

    target_a = jnp.int32(_RANK_A + 1)
    ba, nba, _ = select(target_a)
    bb, _, _ = select(_RANK_A + 2)
    target = target_a - nba

    @plsc.parallel_loop(0, _HALF, _L, unroll=_UNROLL, carry=jnp.int32(0))
    def compact(c0, off):
        b = vbucket_at(c0)
        m = (b == ba) | (b == bb)
        plsc.store_compressed(cand_v.at[pl.ds(off, _L)], bits_at(c0), mask=m)
        return off + jnp.max(plsc.all_reduce_population_count(m))

    cnt = compact
    cand_v[pl.ds(cnt, _L)] = jnp.full((_L,), 0x40000000, jnp.int32)
    n_slices = (cnt + _L - 1) >> 4

    prefix = jnp.int32(0)
    hj = jnp.int32(0)
    for sh, msh in ((22, None), (14, 22), (6, 14), (0, 6)):
        zero_hist()

        @pl.loop(0, n_slices)
        def _(i):
            v = cand_v[pl.ds(i * _L, _L)]
            vf = plsc.bitcast(v, jnp.float32)
            m = (vf * float(_NB)).astype(jnp.int32) == ba
            if msh is not None:
                m = m & ((v >> msh) == prefix)
            bucket = (v >> sh) & (0x3F if sh == 0 else 0xFF)
            plsc.addupdate_scatter(hist_v, [lane_off + bucket], ones, mask=m)

        combine_and_exchange()
        b, nb, hj = select(target)
        prefix = b if msh is None else ((prefix << (msh - sh)) | b)
        target = target - nb

    va = prefix
    cnt_le_a = (_RANK_A + 1 - target) + hj

    minx_v[...] = jnp.full((_L,), _BIG, jnp.int32)

    @pl.loop(0, n_slices)
    def _(i):
        v = cand_v[pl.ds(i * _L, _L)]
        minx_v[...] = jnp.minimum(minx_v[...], jnp.where(v > va, v, _BIG))

    pltpu.sync_copy(minx_v, shared_v.at[s, pl.ds(0, _L)])
    plsc.subcore_barrier()
    pltpu.sync_copy(shared_v.at[s ^ 1, pl.ds(0, _L)], tmp_v.at[pl.ds(0, _L)])
    both = jnp.minimum(minx_v[...], tmp_v[pl.ds(0, _L)])
    min_above = jnp.min(both)

    vb = jnp.where(cnt_le_a >= _RANK_A + 2, va, min_above)
    res = jnp.where(iota == 0, va, jnp.where(iota == 1, vb, 0))
    out_v[...] = plsc.bitcast(res, jnp.float32)

    @pl.when(half == 0)
    def _():
        pltpu.sync_copy(out_v, o_hbm.at[proj])


def _epilogue_body(r_ref, o_ref):
    ceil_v = r_ref[:, 0:1]
    floor_v = r_ref[:, 1:2]
    median = (ceil_v + floor_v) * 0.5
    scaled_min = ceil_v * _MIN_RATIO
    sms = jnp.sum(scaled_min)
    meets_min = (median >= sms).astype(jnp.float32)
    rescaled = _MIN_AMOUNT * (median / sms) * meets_min
    votes = jnp.full((_P, 1), float(_N), jnp.float32)
    elig = jnp.ones((_P, 1), jnp.float32)
    o_ref[...] = jnp.concatenate([votes, median, elig, rescaled], axis=1)


def kernel(x0, x1, x2, x3, x4, x5, x6, x7, x8, x9, x10, x11, x12, x13, x14, x15):
    cp = pltpu.CompilerParams()
    if "needs_layout_passes" in pltpu.CompilerParams.__dataclass_fields__:
        cp = dataclasses.replace(cp, needs_layout_passes=False)
    sc_fn = pl.kernel(
        _sc_body,
        out_type=jax.ShapeDtypeStruct((_P, _L), jnp.float32),
        mesh=plsc.VectorSubcoreMesh(core_axis_name="c", subcore_axis_name="s"),
        compiler_params=cp,
        scratch_types=[
            pltpu.VMEM((_HALF,), jnp.float32),
            pltpu.VMEM((_HALF + 2 * _L,), jnp.int32),
            pltpu.VMEM((_NB * _L,), jnp.int32),
            pltpu.VMEM((_NB,), jnp.int32),
            pltpu.VMEM((_NB,), jnp.int32),
            pltpu.VMEM_SHARED((_L, _NB), jnp.int32),
            pltpu.VMEM((_L,), jnp.int32),
            pltpu.VMEM((_L,), jnp.float32),
        ],
    )
    r = sc_fn(x0, x1, x2, x3, x4, x5, x6, x7, x8, x9, x10, x11, x12, x13,
              x14, x15)

    return pl.pallas_call(
        _epilogue_body,
        out_shape=jax.ShapeDtypeStruct((_P, 4), jnp.float32),
        in_specs=[pl.BlockSpec(memory_space=pltpu.VMEM)],
        out_specs=pl.BlockSpec(memory_space=pltpu.VMEM),
    )(r)
